# baseline (speedup 1.0000x reference)
import functools

import jax
import jax.numpy as jnp
from jax.experimental import pallas as pl
from jax.experimental.pallas import tpu as pltpu

_CR = 128


def _compose_body(nchunks, inner, w1_ref, b1c_ref, w2_ref, mc_ref, w2t_ref):
    c = pl.program_id(0)
    j = pl.program_id(1)
    g = c * inner + j

    @pl.when(j == 0)
    def _():
        w2t_ref[...] = jnp.transpose(w2_ref[...].astype(jnp.bfloat16))

    lhs = jnp.where(g < nchunks, w1_ref[...].astype(jnp.bfloat16), b1c_ref[...])
    mc_ref[...] = jax.lax.dot_general(
        lhs,
        w2t_ref[...],
        dimension_numbers=(((0,), (0,)), ((), ())),
        preferred_element_type=jnp.float32,
    ).astype(mc_ref.dtype)


def _apply_body(isize, x_ref, mc_ref, b2_ref, o_ref):
    xb = x_ref[...].astype(jnp.bfloat16)
    acc = jnp.dot(xb, mc_ref[:isize, :], preferred_element_type=jnp.float32)
    bias = mc_ref[isize : isize + 1, :].astype(jnp.float32) + b2_ref[...]
    o_ref[...] = acc + bias


def kernel(x, w1, b1, w2, b2):
    B, I = x.shape
    H = w1.shape[0]
    O = w2.shape[0]

    nchunks = I // _CR
    nsteps = nchunks + 1
    inner = nsteps
    b1c = jnp.zeros((H, _CR), jnp.bfloat16).at[:, 0].set(b1.astype(jnp.bfloat16))

    mc = pl.pallas_call(
        functools.partial(_compose_body, nchunks, inner),
        out_shape=jax.ShapeDtypeStruct((nsteps * _CR, O), jnp.bfloat16),
        grid=(1, inner),
        in_specs=[
            pl.BlockSpec(
                (H, _CR), lambda c, j: (0, jnp.minimum(c * inner + j, nchunks - 1))
            ),
            pl.BlockSpec((H, _CR), lambda c, j: (0, 0)),
            pl.BlockSpec((O, H), lambda c, j: (0, 0)),
        ],
        out_specs=pl.BlockSpec((_CR, O), lambda c, j: (c * inner + j, 0)),
        scratch_shapes=[pltpu.VMEM((H, O), jnp.bfloat16)],
        compiler_params=pltpu.CompilerParams(
            dimension_semantics=("parallel", "arbitrary"),
        ),
    )(w1, b1c, w2)

    TB = min(512, B)
    out = pl.pallas_call(
        functools.partial(_apply_body, I),
        out_shape=jax.ShapeDtypeStruct((B, O), jnp.float32),
        grid=(B // TB,),
        in_specs=[
            pl.BlockSpec((TB, I), lambda i: (i, 0)),
            pl.BlockSpec((nsteps * _CR, O), lambda i: (0, 0)),
            pl.BlockSpec((1, O), lambda i: (0, 0)),
        ],
        out_specs=pl.BlockSpec((TB, O), lambda i: (i, 0)),
        compiler_params=pltpu.CompilerParams(dimension_semantics=("parallel",)),
    )(x, mc, b2.reshape(1, O).astype(jnp.float32))
    return out


# speedup vs baseline: 1.1528x; 1.1528x over previous
import functools

import jax
import jax.numpy as jnp
from jax.experimental import pallas as pl
from jax.experimental.pallas import tpu as pltpu

_HC = 512
_BC = 128


def _fused_body(nh, x_ref, w1_ref, b1c_ref, w2_ref, b2_ref, o_ref,
                mct_ref, bct_ref, mcb_ref, bias_ref):
    g = pl.program_id(0)

    @pl.when(g < nh)
    def _compose():
        w2c = w2_ref[...].astype(jnp.bfloat16)
        w1c = w1_ref[...].astype(jnp.bfloat16)
        part = jax.lax.dot_general(
            w2c, w1c, dimension_numbers=(((1,), (0,)), ((), ())),
            preferred_element_type=jnp.float32,
        )
        partb = jax.lax.dot_general(
            w2c, b1c_ref[...], dimension_numbers=(((1,), (0,)), ((), ())),
            preferred_element_type=jnp.float32,
        )

        @pl.when(g == 0)
        def _():
            mct_ref[...] = part
            bct_ref[...] = partb

        @pl.when(g > 0)
        def _():
            mct_ref[...] += part
            bct_ref[...] += partb

    @pl.when(g >= nh)
    def _apply():
        @pl.when(g == nh)
        def _():
            mcb_ref[...] = mct_ref[...].astype(jnp.bfloat16)
            bias_ref[...] = jnp.transpose(bct_ref[:, :1]) + b2_ref[...]

        xb = x_ref[...].astype(jnp.bfloat16)
        acc = jax.lax.dot_general(
            xb, mcb_ref[...], dimension_numbers=(((1,), (1,)), ((), ())),
            preferred_element_type=jnp.float32,
        )
        o_ref[...] = acc + bias_ref[...]


def kernel(x, w1, b1, w2, b2):
    B, I = x.shape
    H = w1.shape[0]
    O = w2.shape[0]

    nh = H // _HC
    TB = min(512, B)
    nb = B // TB
    b1c = jnp.zeros((H, _BC), jnp.bfloat16).at[:, 0].set(b1.astype(jnp.bfloat16))

    out = pl.pallas_call(
        functools.partial(_fused_body, nh),
        out_shape=jax.ShapeDtypeStruct((B, O), jnp.float32),
        grid=(nh + nb,),
        in_specs=[
            pl.BlockSpec((TB, I), lambda g: (jnp.maximum(g - nh, 0), 0)),
            pl.BlockSpec((_HC, I), lambda g: (jnp.minimum(g, nh - 1), 0)),
            pl.BlockSpec((_HC, _BC), lambda g: (jnp.minimum(g, nh - 1), 0)),
            pl.BlockSpec((O, _HC), lambda g: (0, jnp.minimum(g, nh - 1))),
            pl.BlockSpec((1, O), lambda g: (0, 0)),
        ],
        out_specs=pl.BlockSpec((TB, O), lambda g: (jnp.maximum(g - nh, 0), 0)),
        scratch_shapes=[
            pltpu.VMEM((O, I), jnp.float32),
            pltpu.VMEM((O, _BC), jnp.float32),
            pltpu.VMEM((O, I), jnp.bfloat16),
            pltpu.VMEM((1, O), jnp.float32),
        ],
        compiler_params=pltpu.CompilerParams(
            dimension_semantics=("arbitrary",),
        ),
    )(x, w1, b1c, w2, b2.reshape(1, O).astype(jnp.float32))
    return out


# speedup vs baseline: 1.2594x; 1.0925x over previous
import functools

import jax
import jax.numpy as jnp
from jax.experimental import pallas as pl
from jax.experimental.pallas import tpu as pltpu

_HC = 1024
_BC = 128


def _fused_body(nh, x_ref, w1_ref, b1c_ref, w2_ref, b2_ref, o_ref,
                mct_ref, bct_ref, mcb_ref, bias_ref):
    g = pl.program_id(0)

    @pl.when(g < nh)
    def _compose():
        w2c = w2_ref[...].astype(jnp.bfloat16)
        w1c = w1_ref[...].astype(jnp.bfloat16)
        part = jax.lax.dot_general(
            w2c, w1c, dimension_numbers=(((1,), (0,)), ((), ())),
            preferred_element_type=jnp.float32,
        )
        partb = jax.lax.dot_general(
            w2c, b1c_ref[...], dimension_numbers=(((1,), (0,)), ((), ())),
            preferred_element_type=jnp.float32,
        )

        @pl.when(g == 0)
        def _():
            mct_ref[...] = part
            bct_ref[...] = partb

        @pl.when(g > 0)
        def _():
            mct_ref[...] += part
            bct_ref[...] += partb

    @pl.when(g >= nh)
    def _apply():
        @pl.when(g == nh)
        def _():
            mcb_ref[...] = mct_ref[...].astype(jnp.bfloat16)
            bias_ref[...] = jnp.transpose(bct_ref[:, :1]) + b2_ref[...]

        xb = x_ref[...].astype(jnp.bfloat16)
        acc = jax.lax.dot_general(
            xb, mcb_ref[...], dimension_numbers=(((1,), (1,)), ((), ())),
            preferred_element_type=jnp.float32,
        )
        o_ref[...] = acc + bias_ref[...]


def kernel(x, w1, b1, w2, b2):
    B, I = x.shape
    H = w1.shape[0]
    O = w2.shape[0]

    nh = H // _HC
    TB = min(1024, B)
    nb = B // TB
    b1c = jnp.zeros((H, _BC), jnp.bfloat16).at[:, 0].set(b1.astype(jnp.bfloat16))

    out = pl.pallas_call(
        functools.partial(_fused_body, nh),
        out_shape=jax.ShapeDtypeStruct((B, O), jnp.float32),
        grid=(nh + nb,),
        in_specs=[
            pl.BlockSpec((TB, I), lambda g: (jnp.maximum(g - nh, 0), 0)),
            pl.BlockSpec((_HC, I), lambda g: (jnp.minimum(g, nh - 1), 0)),
            pl.BlockSpec((_HC, _BC), lambda g: (jnp.minimum(g, nh - 1), 0)),
            pl.BlockSpec((O, _HC), lambda g: (0, jnp.minimum(g, nh - 1))),
            pl.BlockSpec((1, O), lambda g: (0, 0)),
        ],
        out_specs=pl.BlockSpec((TB, O), lambda g: (jnp.maximum(g - nh, 0), 0)),
        scratch_shapes=[
            pltpu.VMEM((O, I), jnp.float32),
            pltpu.VMEM((O, _BC), jnp.float32),
            pltpu.VMEM((O, I), jnp.bfloat16),
            pltpu.VMEM((1, O), jnp.float32),
        ],
        compiler_params=pltpu.CompilerParams(
            dimension_semantics=("arbitrary",),
        ),
    )(x, w1, b1c, w2, b2.reshape(1, O).astype(jnp.float32))
    return out
